# Initial kernel scaffold; baseline (speedup 1.0000x reference)
import jax
import jax.numpy as jnp
from jax.experimental import pallas as pl


def kernel(ids, weight):
    raise NotImplementedError("write your pallas kernel here")


# speedup vs baseline: 2.9813x; 2.9813x over previous
import functools

import jax
import jax.numpy as jnp
from jax import lax
from jax.experimental import pallas as pl
from jax.experimental.pallas import tpu as pltpu
from jax.experimental.pallas import tpu_sc as plsc

CHUNK = 128


@functools.cache
def _make_gather(V, D, B):
    info = plsc.get_sparse_core_info()
    NC, NS = info.num_cores, info.num_subcores
    NW = NC * NS
    assert B % (NW * CHUNK) == 0
    b_per_w = B // NW
    n_ch = b_per_w // CHUNK
    mesh = plsc.VectorSubcoreMesh(core_axis_name="c", subcore_axis_name="s")

    @functools.partial(
        pl.kernel,
        mesh=mesh,
        out_type=jax.ShapeDtypeStruct((B, D), jnp.float32),
        scratch_types=[
            pltpu.VMEM((n_ch, CHUNK), jnp.int32),
            pltpu.VMEM((CHUNK, D), jnp.float32),
            pltpu.SemaphoreType.DMA,
        ],
    )
    def gather_kernel(ids_hbm, table_hbm, out_hbm, idx_v, buf, sem):
        wid = lax.axis_index("s") * NC + lax.axis_index("c")
        base = wid * b_per_w
        pltpu.sync_copy(ids_hbm.at[wid], idx_v)

        def body(j, carry):
            pltpu.async_copy(table_hbm.at[idx_v.at[j]], buf, sem).wait()
            pltpu.sync_copy(buf, out_hbm.at[pl.ds(base + j * CHUNK, CHUNK)])
            return carry

        lax.fori_loop(0, n_ch, body, 0)

    return gather_kernel


def kernel(ids, weight):
    shape = ids.shape
    flat = ids.reshape(-1).astype(jnp.int32)
    B = flat.shape[0]
    V, D = weight.shape
    ids3d = flat.reshape(32, B // (32 * CHUNK), CHUNK)
    out = _make_gather(V, D, B)(ids3d, weight)
    return out.reshape(shape + (D,))


# speedup vs baseline: 3.3368x; 1.1193x over previous
import functools

import jax
import jax.numpy as jnp
from jax import lax
from jax.experimental import pallas as pl
from jax.experimental.pallas import tpu as pltpu
from jax.experimental.pallas import tpu_sc as plsc

CHUNK = 128


@functools.cache
def _make_gather(V, D, B):
    info = plsc.get_sparse_core_info()
    NC, NS = info.num_cores, info.num_subcores
    NW = NC * NS
    assert B % (NW * CHUNK) == 0
    b_per_w = B // NW
    n_ch = b_per_w // CHUNK
    assert n_ch % 2 == 0 and n_ch >= 4
    mesh = plsc.VectorSubcoreMesh(core_axis_name="c", subcore_axis_name="s")

    @functools.partial(
        pl.kernel,
        mesh=mesh,
        out_type=jax.ShapeDtypeStruct((B, D), jnp.float32),
        scratch_types=[
            pltpu.VMEM((n_ch, CHUNK), jnp.int32),
            pltpu.VMEM((CHUNK, D), jnp.float32),
            pltpu.VMEM((CHUNK, D), jnp.float32),
            pltpu.SemaphoreType.DMA,
            pltpu.SemaphoreType.DMA,
            pltpu.SemaphoreType.DMA,
            pltpu.SemaphoreType.DMA,
        ],
    )
    def gather_kernel(ids_hbm, table_hbm, out_hbm, idx_v, buf0, buf1,
                      sg0, sg1, ss0, ss1):
        wid = lax.axis_index("s") * NC + lax.axis_index("c")
        base = wid * b_per_w
        pltpu.sync_copy(ids_hbm.at[wid], idx_v)

        bufs = (buf0, buf1)
        sgs = (sg0, sg1)
        sss = (ss0, ss1)

        def start_gather(j, b):
            pltpu.async_copy(table_hbm.at[idx_v.at[j]], bufs[b], sgs[b])

        def wait_gather(b):
            pltpu.make_async_copy(
                table_hbm.at[idx_v.at[0]], bufs[b], sgs[b]).wait()

        def start_store(j, b):
            pltpu.async_copy(
                bufs[b], out_hbm.at[pl.ds(base + j * CHUNK, CHUNK)], sss[b])

        def wait_store(b):
            pltpu.make_async_copy(
                bufs[b], out_hbm.at[pl.ds(base, CHUNK)], sss[b]).wait()

        start_gather(0, 0)
        start_gather(1, 1)
        wait_gather(0)
        start_store(0, 0)

        def body(g, carry):
            j1 = 2 * g + 1
            wait_store(0)
            start_gather(j1 + 1, 0)
            wait_gather(1)
            start_store(j1, 1)
            j2 = j1 + 1
            wait_store(1)
            start_gather(j2 + 1, 1)
            wait_gather(0)
            start_store(j2, 0)
            return carry

        lax.fori_loop(0, (n_ch - 2) // 2, body, 0)

        wait_store(0)
        wait_gather(1)
        start_store(n_ch - 1, 1)
        wait_store(1)

    return gather_kernel


def kernel(ids, weight):
    shape = ids.shape
    flat = ids.reshape(-1).astype(jnp.int32)
    B = flat.shape[0]
    V, D = weight.shape
    ids3d = flat.reshape(32, B // (32 * CHUNK), CHUNK)
    out = _make_gather(V, D, B)(ids3d, weight)
    return out.reshape(shape + (D,))


# speedup vs baseline: 3.3536x; 1.0050x over previous
import functools

import jax
import jax.numpy as jnp
from jax import lax
from jax.experimental import pallas as pl
from jax.experimental.pallas import tpu as pltpu
from jax.experimental.pallas import tpu_sc as plsc

CHUNK = 128


@functools.cache
def _make_gather(V, D, B):
    info = plsc.get_sparse_core_info()
    NC, NS = info.num_cores, info.num_subcores
    NW = NC * NS
    assert B % (NW * CHUNK) == 0
    b_per_w = B // NW
    n_ch = b_per_w // CHUNK
    NBUF = 4
    assert n_ch >= 8 and (n_ch - 2) % NBUF == 0
    mesh = plsc.VectorSubcoreMesh(core_axis_name="c", subcore_axis_name="s")

    @functools.partial(
        pl.kernel,
        mesh=mesh,
        out_type=jax.ShapeDtypeStruct((B, D), jnp.float32),
        scratch_types=[
            pltpu.VMEM((n_ch, CHUNK), jnp.int32),
            pltpu.VMEM((NBUF, CHUNK, D), jnp.float32),
        ] + [pltpu.SemaphoreType.DMA] * (2 * NBUF),
    )
    def gather_kernel(ids_hbm, table_hbm, out_hbm, idx_v, bufs, *sems):
        sgs = sems[:NBUF]
        sss = sems[NBUF:]
        wid = lax.axis_index("s") * NC + lax.axis_index("c")
        base = wid * b_per_w
        pltpu.sync_copy(ids_hbm.at[wid], idx_v)

        def start_gather(j, b):
            pltpu.async_copy(table_hbm.at[idx_v.at[j]], bufs.at[b], sgs[b])

        def wait_gather(b):
            pltpu.make_async_copy(
                table_hbm.at[idx_v.at[0]], bufs.at[b], sgs[b]).wait()

        def start_store(j, b):
            pltpu.async_copy(
                bufs.at[b], out_hbm.at[pl.ds(base + j * CHUNK, CHUNK)], sss[b])

        def wait_store(b):
            pltpu.make_async_copy(
                bufs.at[b], out_hbm.at[pl.ds(base, CHUNK)], sss[b]).wait()

        start_gather(0, 0)
        start_gather(1, 1)
        for j in (0, 1):
            start_gather(j + 2, (j + 2) % NBUF)
            wait_gather(j % NBUF)
            start_store(j, j % NBUF)

        def body(g, carry):
            j0 = 2 + NBUF * g
            for i in range(NBUF):
                b = (2 + i) % NBUF
                wait_store(i % NBUF)
                start_gather(j0 + i + 2, i % NBUF)
                wait_gather(b)
                start_store(j0 + i, b)
            return carry

        lax.fori_loop(0, (n_ch - 2 - NBUF) // NBUF, body, 0)

        for j in range(n_ch - 4, n_ch):
            b = j % NBUF
            if j + 2 < n_ch:
                wait_store((j + 2) % NBUF)
                start_gather(j + 2, (j + 2) % NBUF)
            wait_gather(b)
            start_store(j, b)
        for j in range(n_ch - 4, n_ch):
            wait_store(j % NBUF)

    return gather_kernel


def kernel(ids, weight):
    shape = ids.shape
    flat = ids.reshape(-1).astype(jnp.int32)
    B = flat.shape[0]
    V, D = weight.shape
    ids3d = flat.reshape(32, B // (32 * CHUNK), CHUNK)
    out = _make_gather(V, D, B)(ids3d, weight)
    return out.reshape(shape + (D,))


# speedup vs baseline: 5.9724x; 1.7809x over previous
import functools

import jax
import jax.numpy as jnp
from jax import lax
from jax.experimental import pallas as pl
from jax.experimental.pallas import tpu as pltpu
from jax.experimental.pallas import tpu_sc as plsc

GRP = 2


@functools.cache
def _make_gather(V, S, D, B):
    info = plsc.get_sparse_core_info()
    NC, NS = info.num_cores, info.num_subcores
    NW = NC * NS
    assert B % (NW * GRP) == 0
    b_per_w = B // NW
    n_ch = b_per_w // GRP
    NBUF = 4
    assert n_ch >= 8
    n_main = (n_ch - 6) // NBUF
    n_back = n_ch - 2 - NBUF * n_main
    mesh = plsc.VectorSubcoreMesh(core_axis_name="c", subcore_axis_name="s")

    @functools.partial(
        pl.kernel,
        mesh=mesh,
        out_type=jax.ShapeDtypeStruct((B, S, D), jnp.float32),
        scratch_types=[
            pltpu.VMEM((n_ch, GRP * S), jnp.int32),
            pltpu.VMEM((NBUF, GRP * S, D), jnp.float32),
        ] + [pltpu.SemaphoreType.DMA] * (2 * NBUF),
    )
    def gather_kernel(ids_hbm, table_hbm, out_hbm, idx_v, bufs, *sems):
        sgs = sems[:NBUF]
        sss = sems[NBUF:]
        wid = lax.axis_index("s") * NC + lax.axis_index("c")
        base = wid * b_per_w
        pltpu.sync_copy(ids_hbm.at[wid], idx_v)

        def start_gather(j, b):
            pltpu.async_copy(table_hbm.at[idx_v.at[j]], bufs.at[b], sgs[b])

        def wait_gather(b):
            pltpu.make_async_copy(
                table_hbm.at[idx_v.at[0]], bufs.at[b], sgs[b]).wait()

        def start_store(j, b):
            pltpu.async_copy(
                bufs.at[b].reshape(GRP, S, D),
                out_hbm.at[pl.ds(base + j * GRP, GRP)], sss[b])

        def wait_store(b):
            pltpu.make_async_copy(
                bufs.at[b].reshape(GRP, S, D),
                out_hbm.at[pl.ds(base, GRP)], sss[b]).wait()

        start_gather(0, 0)
        start_gather(1, 1)
        for j in (0, 1):
            start_gather(j + 2, (j + 2) % NBUF)
            wait_gather(j % NBUF)
            start_store(j, j % NBUF)

        def body(g, carry):
            j0 = 2 + NBUF * g
            for i in range(NBUF):
                b = (2 + i) % NBUF
                wait_store(i % NBUF)
                start_gather(j0 + i + 2, i % NBUF)
                wait_gather(b)
                start_store(j0 + i, b)
            return carry

        lax.fori_loop(0, n_main, body, 0)

        for j in range(n_ch - n_back, n_ch):
            b = j % NBUF
            if j + 2 < n_ch:
                wait_store((j + 2) % NBUF)
                start_gather(j + 2, (j + 2) % NBUF)
            wait_gather(b)
            start_store(j, b)
        for j in range(n_ch - 4, n_ch):
            wait_store(j % NBUF)

    return gather_kernel


def kernel(ids, weight):
    B, S = ids.shape
    V, D = weight.shape
    ids4d = ids.astype(jnp.int32).reshape(32, B // (32 * GRP), GRP * S)
    return _make_gather(V, S, D, B)(ids4d, weight)
